# baseline (speedup 1.0000x reference)
import functools

import jax
import jax.numpy as jnp
from jax import lax
from jax.experimental import pallas as pl
from jax.experimental.pallas import tpu as pltpu
from jax.experimental.pallas import tpu_sc as plsc

N = 100000
DX = 3
DM = 200
K = 100
DOUT = DX + K

NC = 2
NS = 16
NW = NC * NS
ROWS_W = N // NW
R = 125
NBLK = ROWS_W // R
L = 16
NG = (DOUT + L - 1) // L


def _body(x_hbm, eig_hbm, modes_hbm, out_hbm, xbuf, ebuf, obuf, mbuf):
    wid = lax.axis_index("s") * NC + lax.axis_index("c")
    base = wid * ROWS_W

    iota = lax.iota(jnp.int32, L)
    mbuf[pl.ds(96, L)] = jnp.zeros((L,), jnp.int32)
    pltpu.sync_copy(modes_hbm, mbuf.at[pl.ds(0, K)])

    colvecs = [plsc.load_gather(mbuf, [jnp.maximum(iota - DX, 0)])]
    for g in range(1, NG):
        colvecs.append(mbuf[pl.ds(L * g - DX, L)])
    tail_cols = jnp.int32(L * (NG - 1)) + iota
    tail_mask = iota < jnp.int32(DOUT - L * (NG - 1))
    x_mask = iota < DX
    x_cols = jnp.minimum(iota, DX - 1)

    def do_block(b, _):
        r0 = base + b * R
        pltpu.sync_copy(x_hbm.at[pl.ds(r0, R)], xbuf)
        pltpu.sync_copy(eig_hbm.at[pl.ds(r0, R)], ebuf)

        def do_row(r, _):
            rvec = jnp.full((L,), r, jnp.int32)
            ve = plsc.load_gather(ebuf, [rvec, colvecs[0]])
            vx = plsc.load_gather(xbuf, [rvec, x_cols])
            obuf[r, pl.ds(0, L)] = jnp.where(x_mask, vx, ve)
            for g in range(1, NG - 1):
                vals = plsc.load_gather(ebuf, [rvec, colvecs[g]])
                obuf[r, pl.ds(L * g, L)] = vals
            vals = plsc.load_gather(ebuf, [rvec, colvecs[NG - 1]],
                                    mask=tail_mask)
            plsc.store_scatter(obuf, [rvec, tail_cols], vals, mask=tail_mask)
            return ()

        lax.fori_loop(0, R, do_row, ())
        pltpu.sync_copy(obuf, out_hbm.at[pl.ds(r0, R)])
        return ()

    lax.fori_loop(0, NBLK, do_block, ())


@jax.jit
def _run(X, eigenmodes, modes):
    mesh = plsc.VectorSubcoreMesh(core_axis_name="c", subcore_axis_name="s",
                                  num_cores=NC, num_subcores=NS)
    f = pl.kernel(
        _body,
        out_type=jax.ShapeDtypeStruct((N, DOUT), jnp.float32),
        mesh=mesh,
        scratch_types=[
            pltpu.VMEM((R, DX), jnp.float32),
            pltpu.VMEM((R, DM), jnp.float32),
            pltpu.VMEM((R, DOUT), jnp.float32),
            pltpu.VMEM((112,), jnp.int32),
        ],
        compiler_params=pltpu.CompilerParams(use_tc_tiling_on_sc=False,
                                             needs_layout_passes=False),
    )
    return f(X, eigenmodes, modes)


def kernel(X, eigenmodes, modes):
    return _run(X, eigenmodes, modes.astype(jnp.int32))


# speedup vs baseline: 1.1519x; 1.1519x over previous
import functools

import jax
import jax.numpy as jnp
from jax import lax
from jax.experimental import pallas as pl
from jax.experimental.pallas import tpu as pltpu
from jax.experimental.pallas import tpu_sc as plsc

N = 100000
DX = 3
DM = 200
K = 100
DOUT = DX + K

NC = 2
NS = 16
NW = NC * NS
ROWS_W = N // NW
R = 125
NBLK = ROWS_W // R
L = 16
NG = (DOUT + L - 1) // L


def _body(x_hbm, eig_hbm, modes_hbm, out_hbm,
          xb0, xb1, eb0, eb1, ob0, ob1, mbuf,
          sx0, sx1, se0, se1, so0, so1):
    wid = lax.axis_index("s") * NC + lax.axis_index("c")
    base = wid * ROWS_W
    xbufs, ebufs, obufs = (xb0, xb1), (eb0, eb1), (ob0, ob1)
    sxs, ses, sos = (sx0, sx1), (se0, se1), (so0, so1)

    iota = lax.iota(jnp.int32, L)
    mbuf[pl.ds(96, L)] = jnp.zeros((L,), jnp.int32)
    pltpu.sync_copy(modes_hbm, mbuf.at[pl.ds(0, K)])

    colvecs = [plsc.load_gather(mbuf, [jnp.maximum(iota - DX, 0)])]
    for g in range(1, NG):
        colvecs.append(mbuf[pl.ds(L * g - DX, L)])
    tail_cols = jnp.int32(L * (NG - 1)) + iota
    tail_mask = iota < jnp.int32(DOUT - L * (NG - 1))
    x_mask = iota < DX
    x_cols = jnp.minimum(iota, DX - 1)

    def start_in(blk, slot):
        r0 = base + blk * R
        hx = pltpu.async_copy(x_hbm.at[pl.ds(r0, R)], xbufs[slot], sxs[slot])
        he = pltpu.async_copy(eig_hbm.at[pl.ds(r0, R)], ebufs[slot], ses[slot])
        return hx, he

    def compute(xbuf, ebuf, obuf):
        @plsc.parallel_loop(0, R, step=1, unroll=5)
        def _(r):
            rvec = jnp.full((L,), r, jnp.int32)
            ve = plsc.load_gather(ebuf, [rvec, colvecs[0]])
            vx = plsc.load_gather(xbuf, [rvec, x_cols])
            obuf[r, pl.ds(0, L)] = jnp.where(x_mask, vx, ve)
            for g in range(1, NG - 1):
                obuf[r, pl.ds(L * g, L)] = plsc.load_gather(
                    ebuf, [rvec, colvecs[g]])
            vals = plsc.load_gather(ebuf, [rvec, colvecs[NG - 1]],
                                    mask=tail_mask)
            plsc.store_scatter(obuf, [rvec, tail_cols], vals, mask=tail_mask)

    pend_in = {0: start_in(0, 0)}
    pend_out = {}
    for blk in range(NBLK):
        slot = blk % 2
        if blk + 1 < NBLK:
            pend_in[blk + 1] = start_in(blk + 1, 1 - slot)
        hx, he = pend_in.pop(blk)
        hx.wait()
        he.wait()
        if blk >= 2:
            pend_out.pop(blk - 2).wait()
        compute(xbufs[slot], ebufs[slot], obufs[slot])
        r0 = base + blk * R
        pend_out[blk] = pltpu.async_copy(
            obufs[slot], out_hbm.at[pl.ds(r0, R)], sos[slot])
    for blk in (NBLK - 2, NBLK - 1):
        pend_out.pop(blk).wait()


@jax.jit
def _run(X, eigenmodes, modes):
    mesh = plsc.VectorSubcoreMesh(core_axis_name="c", subcore_axis_name="s",
                                  num_cores=NC, num_subcores=NS)
    f = pl.kernel(
        _body,
        out_type=jax.ShapeDtypeStruct((N, DOUT), jnp.float32),
        mesh=mesh,
        scratch_types=[
            pltpu.VMEM((R, DX), jnp.float32),
            pltpu.VMEM((R, DX), jnp.float32),
            pltpu.VMEM((R, DM), jnp.float32),
            pltpu.VMEM((R, DM), jnp.float32),
            pltpu.VMEM((R, DOUT), jnp.float32),
            pltpu.VMEM((R, DOUT), jnp.float32),
            pltpu.VMEM((112,), jnp.int32),
            pltpu.SemaphoreType.DMA,
            pltpu.SemaphoreType.DMA,
            pltpu.SemaphoreType.DMA,
            pltpu.SemaphoreType.DMA,
            pltpu.SemaphoreType.DMA,
            pltpu.SemaphoreType.DMA,
        ],
        compiler_params=pltpu.CompilerParams(use_tc_tiling_on_sc=False,
                                             needs_layout_passes=False),
    )
    return f(X, eigenmodes, modes)


def kernel(X, eigenmodes, modes):
    return _run(X, eigenmodes, modes.astype(jnp.int32))
